# Initial kernel scaffold; baseline (speedup 1.0000x reference)
import functools

import jax
import jax.numpy as jnp
from jax import lax
from jax.experimental import pallas as pl
from jax.experimental.pallas import tpu as pltpu
from jax.experimental.pallas import tpu_sc as plsc

USER = 20000
ITEM = 30000
NN = USER + ITEM
D = 64
E = 800000
B = 4096
NEG = 16

NC, NS = 2, 16
NW = NC * NS
EPT = E // NW
CH = 128
PAD_EPT = 25600
NCHUNK = PAD_EPT // CH
ACC_ROWS = 30720
ROWS_PT = ACC_ROWS // NS
SCRAP = ACC_ROWS - 1
WB = 80

NG = 2 * (B + B + B * NEG)
GPT = NG // NW
GCH = GPT // CH

_MESH = plsc.VectorSubcoreMesh(core_axis_name="c", subcore_axis_name="s")
_f32 = jnp.float32


def _zero16():
    return jnp.zeros((16,), _f32)


@functools.partial(
    pl.kernel,
    mesh=_MESH,
    out_type=jax.ShapeDtypeStruct((NN,), _f32),
    scratch_types=[
        pltpu.VMEM((CH,), jnp.int32),
        pltpu.VMEM((ACC_ROWS,), _f32),
        pltpu.VMEM((ROWS_PT,), _f32),
        pltpu.VMEM((ROWS_PT,), _f32),
        pltpu.VMEM_SHARED((NS, ACC_ROWS), _f32),
    ],
)
def _hist(rowp, deg_out, row_v, hist_v, part_v, sum_v, stage):
    cid = lax.axis_index("c")
    sid = lax.axis_index("s")
    ebase = (cid * NS + sid) * PAD_EPT

    @pl.loop(0, ACC_ROWS // 16)
    def _(i):
        hist_v[pl.ds(i * 16, 16)] = _zero16()

    @pl.loop(0, NCHUNK)
    def _(c):
        pltpu.sync_copy(rowp.at[pl.ds(ebase + c * CH, CH)], row_v)
        for j in range(CH // 16):
            idx16 = row_v[pl.ds(j * 16, 16)]
            plsc.addupdate_scatter(hist_v, [idx16], jnp.ones((16,), _f32))

    pltpu.sync_copy(hist_v, stage.at[sid])
    plsc.subcore_barrier()

    @pl.loop(0, ROWS_PT // 16)
    def _(i):
        sum_v[pl.ds(i * 16, 16)] = _zero16()

    @pl.loop(0, NS)
    def _(p):
        pltpu.sync_copy(stage.at[p, pl.ds(sid * ROWS_PT, ROWS_PT)], part_v)

        @pl.loop(0, ROWS_PT // 16)
        def _(i):
            sl = pl.ds(i * 16, 16)
            sum_v[sl] = sum_v[sl] + part_v[sl]

    limit = jnp.where(cid == 0, USER, ITEM)
    gbase = cid * USER

    @pl.loop(0, ROWS_PT // WB)
    def _(j):
        r0 = sid * ROWS_PT + j * WB

        @pl.when(r0 < limit)
        def _():
            pltpu.sync_copy(sum_v.at[pl.ds(j * WB, WB)],
                            deg_out.at[pl.ds(gbase + r0, WB)])


@functools.partial(
    pl.kernel,
    mesh=_MESH,
    out_type=jax.ShapeDtypeStruct((NN, D), _f32),
    scratch_types=[
        pltpu.VMEM((CH,), jnp.int32),
        pltpu.VMEM((CH,), jnp.int32),
        pltpu.VMEM((CH, D), _f32),
        pltpu.VMEM((CH, D), _f32),
        pltpu.VMEM_SHARED((ACC_ROWS, D), _f32),
    ],
)
def _prop(colp, rowp, t_in, u_out, col_v, row_v, rows_v, zero_v, acc):
    cid = lax.axis_index("c")
    sid = lax.axis_index("s")
    ebase = (cid * NS + sid) * PAD_EPT

    @pl.loop(0, CH)
    def _(r):
        for j in range(D // 16):
            zero_v[r, pl.ds(j * 16, 16)] = _zero16()

    @pl.loop(0, ROWS_PT // CH)
    def _(j):
        pltpu.sync_copy(zero_v, acc.at[pl.ds(sid * ROWS_PT + j * CH, CH)])

    plsc.subcore_barrier()

    @pl.loop(0, NCHUNK)
    def _(c):
        off = ebase + c * CH
        pltpu.sync_copy(colp.at[pl.ds(off, CH)], col_v)
        pltpu.sync_copy(rowp.at[pl.ds(off, CH)], row_v)
        pltpu.sync_copy(t_in.at[col_v], rows_v)
        pltpu.sync_copy(rows_v, acc.at[row_v], add=True)

    plsc.subcore_barrier()

    limit = jnp.where(cid == 0, USER, ITEM)
    gbase = cid * USER

    @pl.loop(0, ROWS_PT // WB)
    def _(j):
        r0 = sid * ROWS_PT + j * WB

        @pl.when(r0 < limit)
        def _():
            pltpu.sync_copy(acc.at[pl.ds(r0, WB)],
                            u_out.at[pl.ds(gbase + r0, WB)])


@functools.partial(
    pl.kernel,
    mesh=_MESH,
    out_type=jax.ShapeDtypeStruct((NG, D), _f32),
    scratch_types=[
        pltpu.VMEM((CH,), jnp.int32),
        pltpu.VMEM((CH, D), _f32),
    ],
)
def _gath(big, cat_idx, out, idx_v, rows_v):
    cid = lax.axis_index("c")
    sid = lax.axis_index("s")
    base = (cid * NS + sid) * GPT

    @pl.loop(0, GCH)
    def _(c):
        off = base + c * CH
        pltpu.sync_copy(cat_idx.at[pl.ds(off, CH)], idx_v)
        pltpu.sync_copy(big.at[idx_v], rows_v)
        pltpu.sync_copy(rows_v, out.at[pl.ds(off, CH)])


_RB = 2000


def _s0_body(deg_ref, e_ref, t_ref, a_ref, asq_ref):
    dg = jnp.maximum(deg_ref[...], 1.0)
    asq = 1.0 / dg
    a = lax.rsqrt(dg)
    a_ref[...] = a
    asq_ref[...] = asq
    t_ref[...] = e_ref[...] * a


def _scale_init(deg, e0):
    return pl.pallas_call(
        _s0_body,
        grid=(NN // _RB,),
        in_specs=[
            pl.BlockSpec((_RB, 1), lambda i: (i, 0)),
            pl.BlockSpec((_RB, D), lambda i: (i, 0)),
        ],
        out_specs=[
            pl.BlockSpec((_RB, D), lambda i: (i, 0)),
            pl.BlockSpec((_RB, 1), lambda i: (i, 0)),
            pl.BlockSpec((_RB, 1), lambda i: (i, 0)),
        ],
        out_shape=[
            jax.ShapeDtypeStruct((NN, D), _f32),
            jax.ShapeDtypeStruct((NN, 1), _f32),
            jax.ShapeDtypeStruct((NN, 1), _f32),
        ],
    )(deg, e0)


def _smid_body(u_ref, asq_ref, t_ref):
    t_ref[...] = u_ref[...] * asq_ref[...]


def _scale_mid(u, asq):
    return pl.pallas_call(
        _smid_body,
        grid=(NN // _RB,),
        in_specs=[
            pl.BlockSpec((_RB, D), lambda i: (i, 0)),
            pl.BlockSpec((_RB, 1), lambda i: (i, 0)),
        ],
        out_specs=pl.BlockSpec((_RB, D), lambda i: (i, 0)),
        out_shape=jax.ShapeDtypeStruct((NN, D), _f32),
    )(u, asq)


def _mean_body(e_ref, u1_ref, u2_ref, u3_ref, a_ref, o_ref):
    s = u1_ref[...] + u2_ref[...] + u3_ref[...]
    o_ref[...] = 0.25 * (e_ref[...] + a_ref[...] * s)


def _mean(e0, u1, u2, u3, a):
    spec = pl.BlockSpec((_RB, D), lambda i: (i, 0))
    return pl.pallas_call(
        _mean_body,
        grid=(NN // _RB,),
        in_specs=[spec, spec, spec, spec, pl.BlockSpec((_RB, 1), lambda i: (i, 0))],
        out_specs=spec,
        out_shape=jax.ShapeDtypeStruct((NN, D), _f32),
    )(e0, u1, u2, u3, a)


_BB = 512


def _score_body(u_ref, p_ref, n_ref, uw_ref, pw_ref, nw_ref,
                ps_ref, ns_ref, reg_ref):
    i = pl.program_id(0)
    u = u_ref[...]
    ps_ref[...] = jnp.sum(u * p_ref[...], axis=1, keepdims=True)
    ns_ref[...] = jnp.sum(u[:, None, :] * n_ref[...], axis=-1)
    part = (jnp.sum(uw_ref[...] ** 2) + jnp.sum(pw_ref[...] ** 2)
            + jnp.sum(nw_ref[...] ** 2)) * (1.0 / B)

    @pl.when(i == 0)
    def _():
        reg_ref[0, 0] = part

    @pl.when(i > 0)
    def _():
        reg_ref[0, 0] = reg_ref[0, 0] + part


def _score(uvec, pvec, nvec, uw, pw, nw):
    v2 = pl.BlockSpec((_BB, D), lambda i: (i, 0))
    v3 = pl.BlockSpec((_BB, NEG, D), lambda i: (i, 0, 0))
    return pl.pallas_call(
        _score_body,
        grid=(B // _BB,),
        in_specs=[v2, v2, v3, v2, v2, v3],
        out_specs=[
            pl.BlockSpec((_BB, 1), lambda i: (i, 0)),
            pl.BlockSpec((_BB, NEG), lambda i: (i, 0)),
            pl.BlockSpec((1, 1), lambda i: (0, 0)),
        ],
        out_shape=[
            jax.ShapeDtypeStruct((B, 1), _f32),
            jax.ShapeDtypeStruct((B, NEG), _f32),
            jax.ShapeDtypeStruct((1, 1), _f32),
        ],
    )(uvec, pvec, nvec, uw, pw, nw)


def kernel(U_weight, I_weight, edge_vals, user, pos_item, neg_item,
           edge_row, edge_col):
    del edge_vals
    all_emb0 = jnp.concatenate([U_weight, I_weight], axis=0)

    row2 = edge_row.reshape(NW, EPT)
    col2 = edge_col.reshape(NW, EPT)
    reg_off = jnp.where(jnp.arange(NW)[:, None] < NS, 0, USER).astype(jnp.int32)
    rowloc = row2 - reg_off
    pad_row = jnp.full((NW, PAD_EPT - EPT), SCRAP, jnp.int32)
    pad_col = jnp.zeros((NW, PAD_EPT - EPT), jnp.int32)
    rowp = jnp.concatenate([rowloc, pad_row], axis=1).reshape(-1)
    colp = jnp.concatenate([col2, pad_col], axis=1).reshape(-1)

    deg = _hist(rowp)
    t0, a, asq = _scale_init(deg.reshape(NN, 1), all_emb0)
    u1 = _prop(colp, rowp, t0)
    t1 = _scale_mid(u1, asq)
    u2 = _prop(colp, rowp, t1)
    t2 = _scale_mid(u2, asq)
    u3 = _prop(colp, rowp, t2)
    light = _mean(all_emb0, u1, u2, u3, a)

    big = jnp.concatenate([light, all_emb0], axis=0)
    negf = neg_item.reshape(-1)
    cat_idx = jnp.concatenate([
        user, pos_item + USER, negf + USER,
        user + NN, pos_item + NN + USER, negf + NN + USER,
    ]).astype(jnp.int32)
    g = _gath(big, cat_idx)

    uvec = g[0:B]
    pvec = g[B:2 * B]
    nvec = g[2 * B:2 * B + B * NEG].reshape(B, NEG, D)
    o = 2 * B + B * NEG
    uw = g[o:o + B]
    pw = g[o + B:o + 2 * B]
    nw = g[o + 2 * B:o + 2 * B + B * NEG].reshape(B, NEG, D)

    ps, ns, reg = _score(uvec, pvec, nvec, uw, pw, nw)
    return ps, ns, reg[0, 0]


# speedup vs baseline: 2.9546x; 2.9546x over previous
import dataclasses
import functools

import jax
import jax.numpy as jnp
from jax import lax
from jax.experimental import pallas as pl
from jax.experimental.pallas import tpu as pltpu
from jax.experimental.pallas import tpu_sc as plsc

USER = 20000
ITEM = 30000
NN = USER + ITEM
D = 64
HD = D // 2
E = 800000
B = 4096
NEG = 16

NC, NS = 2, 16
NW = NC * NS
EPT = E // NW
CH = 128
PAD_EPT = 25600
E_PAD = NW * PAD_EPT
HCHUNK = PAD_EPT // CH
PCHUNK = 2 * HCHUNK

ACC_ROWS = 51200
ROWS_PT = ACC_ROWS // NS
SCRAP = ACC_ROWS - 1
WB = 80

NG = 2 * (B + B + B * NEG)
GPT = NG // NW
GCH = GPT // CH

_MESH = plsc.VectorSubcoreMesh(core_axis_name="c", subcore_axis_name="s")
_f32 = jnp.float32

_SC_CP = pltpu.CompilerParams()
if "needs_layout_passes" in pltpu.CompilerParams.__dataclass_fields__:
    _SC_CP = dataclasses.replace(_SC_CP, needs_layout_passes=False)
if "use_tc_tiling_on_sc" in pltpu.CompilerParams.__dataclass_fields__:
    _SC_CP = dataclasses.replace(_SC_CP, use_tc_tiling_on_sc=False)


def _zero16():
    return jnp.zeros((16,), _f32)


@functools.partial(
    pl.kernel,
    mesh=_MESH,
    out_type=jax.ShapeDtypeStruct((NN,), _f32),
    scratch_types=[
        pltpu.VMEM((CH,), jnp.int32),
        pltpu.VMEM((ACC_ROWS,), _f32),
        pltpu.VMEM((ROWS_PT,), _f32),
        pltpu.VMEM((ROWS_PT,), _f32),
        pltpu.VMEM_SHARED((NS, ACC_ROWS), _f32),
    ],
    compiler_params=_SC_CP,
)
def _hist(rowp, deg_out, row_v, hist_v, part_v, sum_v, stage):
    cid = lax.axis_index("c")
    sid = lax.axis_index("s")
    ebase = (cid * NS + sid) * PAD_EPT

    @pl.loop(0, ACC_ROWS // 16)
    def _(i):
        hist_v[pl.ds(i * 16, 16)] = _zero16()

    @pl.loop(0, HCHUNK)
    def _(c):
        pltpu.sync_copy(rowp.at[pl.ds(ebase + c * CH, CH)], row_v)
        for j in range(CH // 16):
            idx16 = row_v[pl.ds(j * 16, 16)]
            plsc.addupdate_scatter(hist_v, [idx16], jnp.ones((16,), _f32))

    pltpu.sync_copy(hist_v, stage.at[sid])
    plsc.subcore_barrier()

    @pl.loop(0, ROWS_PT // 16)
    def _(i):
        sum_v[pl.ds(i * 16, 16)] = _zero16()

    @pl.loop(0, NS)
    def _(p):
        pltpu.sync_copy(stage.at[p, pl.ds(sid * ROWS_PT, ROWS_PT)], part_v)

        @pl.loop(0, ROWS_PT // 16)
        def _(i):
            sl = pl.ds(i * 16, 16)
            sum_v[sl] = sum_v[sl] + part_v[sl]

    lo = cid * USER
    hi = jnp.where(cid == 0, USER, NN)

    @pl.loop(0, ROWS_PT // WB)
    def _(j):
        r0 = sid * ROWS_PT + j * WB

        @pl.when(jnp.logical_and(r0 >= lo, r0 < hi))
        def _():
            pltpu.sync_copy(sum_v.at[pl.ds(j * WB, WB)],
                            deg_out.at[pl.ds(r0, WB)])


@functools.partial(
    pl.kernel,
    mesh=_MESH,
    out_type=[
        jax.ShapeDtypeStruct((NN, HD), _f32),
        jax.ShapeDtypeStruct((NN, HD), _f32),
    ],
    scratch_types=[
        pltpu.VMEM((CH,), jnp.int32),
        pltpu.VMEM((CH,), jnp.int32),
        pltpu.VMEM((CH, HD), _f32),
        pltpu.VMEM((CH, HD), _f32),
        pltpu.VMEM_SHARED((ACC_ROWS, HD), _f32),
    ],
    compiler_params=_SC_CP,
)
def _prop(colp, rowp, t_lo, t_hi, u_lo, u_hi, col_v, row_v, rows_v, zero_v,
          acc):
    cid = lax.axis_index("c")
    sid = lax.axis_index("s")
    ebase = sid * (2 * PAD_EPT)

    @pl.loop(0, CH)
    def _(r):
        for j in range(HD // 16):
            zero_v[r, pl.ds(j * 16, 16)] = _zero16()

    @pl.loop(0, ROWS_PT // CH)
    def _(j):
        pltpu.sync_copy(zero_v, acc.at[pl.ds(sid * ROWS_PT + j * CH, CH)])

    plsc.subcore_barrier()

    def edge_pass(t_ref):
        @pl.loop(0, PCHUNK)
        def _(c):
            off = ebase + c * CH
            pltpu.sync_copy(colp.at[pl.ds(off, CH)], col_v)
            pltpu.sync_copy(rowp.at[pl.ds(off, CH)], row_v)
            pltpu.sync_copy(t_ref.at[col_v], rows_v)
            pltpu.sync_copy(rows_v, acc.at[row_v], add=True)

    @pl.when(cid == 0)
    def _():
        edge_pass(t_lo)

    @pl.when(cid == 1)
    def _():
        edge_pass(t_hi)

    plsc.subcore_barrier()

    def writeback(u_ref):
        @pl.loop(0, ROWS_PT // WB)
        def _(j):
            r0 = sid * ROWS_PT + j * WB

            @pl.when(r0 < NN)
            def _():
                pltpu.sync_copy(acc.at[pl.ds(r0, WB)], u_ref.at[pl.ds(r0, WB)])

    @pl.when(cid == 0)
    def _():
        writeback(u_lo)

    @pl.when(cid == 1)
    def _():
        writeback(u_hi)


@functools.partial(
    pl.kernel,
    mesh=_MESH,
    out_type=jax.ShapeDtypeStruct((NG, D), _f32),
    scratch_types=[
        pltpu.VMEM((CH,), jnp.int32),
        pltpu.VMEM((CH, D), _f32),
    ],
    compiler_params=_SC_CP,
)
def _gath(big, cat_idx, out, idx_v, rows_v):
    cid = lax.axis_index("c")
    sid = lax.axis_index("s")
    base = (cid * NS + sid) * GPT

    @pl.loop(0, GCH)
    def _(c):
        off = base + c * CH
        pltpu.sync_copy(cat_idx.at[pl.ds(off, CH)], idx_v)
        pltpu.sync_copy(big.at[idx_v], rows_v)
        pltpu.sync_copy(rows_v, out.at[pl.ds(off, CH)])


_RB = 2000


def _s0_body(deg_ref, e_ref, tlo_ref, thi_ref, a_ref, asq_ref):
    dg = jnp.maximum(deg_ref[...], 1.0)
    asq = 1.0 / dg
    a = lax.rsqrt(dg)
    a_ref[...] = a
    asq_ref[...] = asq
    t = e_ref[...] * a
    tlo_ref[...] = t[:, :HD]
    thi_ref[...] = t[:, HD:]


def _scale_init(deg, e0):
    return pl.pallas_call(
        _s0_body,
        grid=(NN // _RB,),
        in_specs=[
            pl.BlockSpec((_RB, 1), lambda i: (i, 0)),
            pl.BlockSpec((_RB, D), lambda i: (i, 0)),
        ],
        out_specs=[
            pl.BlockSpec((_RB, HD), lambda i: (i, 0)),
            pl.BlockSpec((_RB, HD), lambda i: (i, 0)),
            pl.BlockSpec((_RB, 1), lambda i: (i, 0)),
            pl.BlockSpec((_RB, 1), lambda i: (i, 0)),
        ],
        out_shape=[
            jax.ShapeDtypeStruct((NN, HD), _f32),
            jax.ShapeDtypeStruct((NN, HD), _f32),
            jax.ShapeDtypeStruct((NN, 1), _f32),
            jax.ShapeDtypeStruct((NN, 1), _f32),
        ],
    )(deg, e0)


def _smid_body(ulo_ref, uhi_ref, asq_ref, tlo_ref, thi_ref):
    asq = asq_ref[...]
    tlo_ref[...] = ulo_ref[...] * asq
    thi_ref[...] = uhi_ref[...] * asq


def _scale_mid(u_lo, u_hi, asq):
    h = pl.BlockSpec((_RB, HD), lambda i: (i, 0))
    return pl.pallas_call(
        _smid_body,
        grid=(NN // _RB,),
        in_specs=[h, h, pl.BlockSpec((_RB, 1), lambda i: (i, 0))],
        out_specs=[h, h],
        out_shape=[
            jax.ShapeDtypeStruct((NN, HD), _f32),
            jax.ShapeDtypeStruct((NN, HD), _f32),
        ],
    )(u_lo, u_hi, asq)


def _mean_body(e_ref, l1, h1, l2, h2, l3, h3, a_ref, o_ref):
    a = a_ref[...]
    slo = l1[...] + l2[...] + l3[...]
    shi = h1[...] + h2[...] + h3[...]
    o_ref[:, :HD] = 0.25 * (e_ref[:, :HD] + a * slo)
    o_ref[:, HD:] = 0.25 * (e_ref[:, HD:] + a * shi)


def _mean(e0, us, a):
    h = pl.BlockSpec((_RB, HD), lambda i: (i, 0))
    f = pl.BlockSpec((_RB, D), lambda i: (i, 0))
    return pl.pallas_call(
        _mean_body,
        grid=(NN // _RB,),
        in_specs=[f, h, h, h, h, h, h, pl.BlockSpec((_RB, 1), lambda i: (i, 0))],
        out_specs=f,
        out_shape=jax.ShapeDtypeStruct((NN, D), _f32),
    )(e0, *us, a)


_BB = 512


def _score_body(u_ref, p_ref, n_ref, uw_ref, pw_ref, nw_ref,
                ps_ref, ns_ref, reg_ref):
    i = pl.program_id(0)
    u = u_ref[...]
    ps_ref[...] = jnp.sum(u * p_ref[...], axis=1, keepdims=True)
    ns_ref[...] = jnp.sum(u[:, None, :] * n_ref[...], axis=-1)
    part = (jnp.sum(uw_ref[...] ** 2) + jnp.sum(pw_ref[...] ** 2)
            + jnp.sum(nw_ref[...] ** 2)) * (1.0 / B)

    @pl.when(i == 0)
    def _():
        reg_ref[...] = jnp.zeros((1, 1), _f32)

    reg_ref[...] = reg_ref[...] + part


def _score(uvec, pvec, nvec, uw, pw, nw):
    v2 = pl.BlockSpec((_BB, D), lambda i: (i, 0))
    v3 = pl.BlockSpec((_BB, NEG, D), lambda i: (i, 0, 0))
    return pl.pallas_call(
        _score_body,
        grid=(B // _BB,),
        in_specs=[v2, v2, v3, v2, v2, v3],
        out_specs=[
            pl.BlockSpec((_BB, 1), lambda i: (i, 0)),
            pl.BlockSpec((_BB, NEG), lambda i: (i, 0)),
            pl.BlockSpec((1, 1), lambda i: (0, 0)),
        ],
        out_shape=[
            jax.ShapeDtypeStruct((B, 1), _f32),
            jax.ShapeDtypeStruct((B, NEG), _f32),
            jax.ShapeDtypeStruct((1, 1), _f32),
        ],
    )(uvec, pvec, nvec, uw, pw, nw)


def kernel(U_weight, I_weight, edge_vals, user, pos_item, neg_item,
           edge_row, edge_col):
    del edge_vals
    all_emb0 = jnp.concatenate([U_weight, I_weight], axis=0)

    row2 = edge_row.reshape(NW, EPT)
    col2 = edge_col.reshape(NW, EPT)
    pad_row = jnp.full((NW, PAD_EPT - EPT), SCRAP, jnp.int32)
    pad_col = jnp.zeros((NW, PAD_EPT - EPT), jnp.int32)
    rowp = jnp.concatenate([row2, pad_row], axis=1).reshape(-1)
    colp = jnp.concatenate([col2, pad_col], axis=1).reshape(-1)

    deg = _hist(rowp)
    t_lo, t_hi, a, asq = _scale_init(deg.reshape(NN, 1), all_emb0)
    u1 = _prop(colp, rowp, t_lo, t_hi)
    t_lo, t_hi = _scale_mid(u1[0], u1[1], asq)
    u2 = _prop(colp, rowp, t_lo, t_hi)
    t_lo, t_hi = _scale_mid(u2[0], u2[1], asq)
    u3 = _prop(colp, rowp, t_lo, t_hi)
    light = _mean(all_emb0, (u1[0], u1[1], u2[0], u2[1], u3[0], u3[1]), a)

    big = jnp.concatenate([light, all_emb0], axis=0)
    negf = neg_item.reshape(-1)
    cat_idx = jnp.concatenate([
        user, pos_item + USER, negf + USER,
        user + NN, pos_item + NN + USER, negf + NN + USER,
    ]).astype(jnp.int32)
    g = _gath(big, cat_idx)

    uvec = g[0:B]
    pvec = g[B:2 * B]
    nvec = g[2 * B:2 * B + B * NEG].reshape(B, NEG, D)
    o = 2 * B + B * NEG
    uw = g[o:o + B]
    pw = g[o + B:o + 2 * B]
    nw = g[o + 2 * B:o + 2 * B + B * NEG].reshape(B, NEG, D)

    ps, ns, reg = _score(uvec, pvec, nvec, uw, pw, nw)
    return ps, ns, reg[0, 0]


# speedup vs baseline: 4.8662x; 1.6470x over previous
import dataclasses
import functools

import jax
import jax.numpy as jnp
from jax import lax
from jax.experimental import pallas as pl
from jax.experimental.pallas import tpu as pltpu
from jax.experimental.pallas import tpu_sc as plsc

USER = 20000
ITEM = 30000
NN = USER + ITEM
D = 64
HD = D // 2
E = 800000
B = 4096
NEG = 16

NC, NS = 2, 16
NW = NC * NS
EPT = E // NW
CH = 128
PAD_EPT = 25600
E_PAD = NW * PAD_EPT
HCHUNK = PAD_EPT // CH
PCHUNK = 2 * HCHUNK

ACC_ROWS = 51200
ROWS_PT = ACC_ROWS // NS
SCRAP = ACC_ROWS - 1
WB = 80

NG = 2 * (B + B + B * NEG)
GPT = NG // NW
GCH = GPT // CH

_MESH = plsc.VectorSubcoreMesh(core_axis_name="c", subcore_axis_name="s")
_f32 = jnp.float32

_SC_CP = pltpu.CompilerParams()
if "needs_layout_passes" in pltpu.CompilerParams.__dataclass_fields__:
    _SC_CP = dataclasses.replace(_SC_CP, needs_layout_passes=False)
if "use_tc_tiling_on_sc" in pltpu.CompilerParams.__dataclass_fields__:
    _SC_CP = dataclasses.replace(_SC_CP, use_tc_tiling_on_sc=False)


def _zero16():
    return jnp.zeros((16,), _f32)


@functools.partial(
    pl.kernel,
    mesh=_MESH,
    out_type=jax.ShapeDtypeStruct((NN,), _f32),
    scratch_types=[
        pltpu.VMEM((CH,), jnp.int32),
        pltpu.VMEM((ACC_ROWS,), _f32),
        pltpu.VMEM((ROWS_PT,), _f32),
        pltpu.VMEM((ROWS_PT,), _f32),
        pltpu.VMEM_SHARED((NS, ACC_ROWS), _f32),
    ],
    compiler_params=_SC_CP,
)
def _hist(rowp, deg_out, row_v, hist_v, part_v, sum_v, stage):
    cid = lax.axis_index("c")
    sid = lax.axis_index("s")
    ebase = (cid * NS + sid) * PAD_EPT

    @pl.loop(0, ACC_ROWS // 16)
    def _(i):
        hist_v[pl.ds(i * 16, 16)] = _zero16()

    @pl.loop(0, HCHUNK)
    def _(c):
        pltpu.sync_copy(rowp.at[pl.ds(ebase + c * CH, CH)], row_v)
        for j in range(CH // 16):
            idx16 = row_v[pl.ds(j * 16, 16)]
            plsc.addupdate_scatter(hist_v, [idx16], jnp.ones((16,), _f32))

    pltpu.sync_copy(hist_v, stage.at[sid])
    plsc.subcore_barrier()

    @pl.loop(0, ROWS_PT // 16)
    def _(i):
        sum_v[pl.ds(i * 16, 16)] = _zero16()

    @pl.loop(0, NS)
    def _(p):
        pltpu.sync_copy(stage.at[p, pl.ds(sid * ROWS_PT, ROWS_PT)], part_v)

        @pl.loop(0, ROWS_PT // 16)
        def _(i):
            sl = pl.ds(i * 16, 16)
            sum_v[sl] = sum_v[sl] + part_v[sl]

    lo = cid * USER
    hi = jnp.where(cid == 0, USER, NN)

    @pl.loop(0, ROWS_PT // WB)
    def _(j):
        r0 = sid * ROWS_PT + j * WB

        @pl.when(jnp.logical_and(r0 >= lo, r0 < hi))
        def _():
            pltpu.sync_copy(sum_v.at[pl.ds(j * WB, WB)],
                            deg_out.at[pl.ds(r0, WB)])


NB = 4
NGRP = PCHUNK // NB


@functools.partial(
    pl.kernel,
    mesh=_MESH,
    out_type=[
        jax.ShapeDtypeStruct((NN, HD), _f32),
        jax.ShapeDtypeStruct((NN, HD), _f32),
    ],
    scratch_types=[
        pltpu.VMEM((NB, 2, CH), jnp.int32),
        pltpu.VMEM((NB, CH, HD), _f32),
        pltpu.VMEM((CH, HD), _f32),
        pltpu.VMEM_SHARED((ACC_ROWS, HD), _f32),
        pltpu.SemaphoreType.DMA,
        pltpu.SemaphoreType.DMA,
        pltpu.SemaphoreType.DMA,
    ],
    compiler_params=_SC_CP,
)
def _prop(idxp, t_lo, t_hi, u_lo, u_hi, ibuf, rbuf, zero_v, acc,
          isem, gsem, ssem):
    cid = lax.axis_index("c")
    sid = lax.axis_index("s")
    cbase = sid * PCHUNK

    @pl.loop(0, CH)
    def _(r):
        for j in range(HD // 16):
            zero_v[r, pl.ds(j * 16, 16)] = _zero16()

    zcps = []
    for j in range(ROWS_PT // CH):
        zcps.append(pltpu.async_copy(
            zero_v, acc.at[pl.ds(sid * ROWS_PT + j * CH, CH)], ssem))
    for cp in zcps:
        cp.wait()

    plsc.subcore_barrier()

    def edge_pass(t_ref):
        @pl.loop(0, NGRP)
        def _(g):
            c0 = cbase + g * NB
            icps = [pltpu.async_copy(idxp.at[c0 + b], ibuf.at[b], isem)
                    for b in range(NB)]
            for cp in icps:
                cp.wait()
            gcps = [pltpu.async_copy(t_ref.at[ibuf.at[b, 0]], rbuf.at[b], gsem)
                    for b in range(NB)]
            for cp in gcps:
                cp.wait()
            scps = [pltpu.async_copy(rbuf.at[b], acc.at[ibuf.at[b, 1]], ssem,
                                     add=True)
                    for b in range(NB)]
            for cp in scps:
                cp.wait()

    @pl.when(cid == 0)
    def _():
        edge_pass(t_lo)

    @pl.when(cid == 1)
    def _():
        edge_pass(t_hi)

    plsc.subcore_barrier()

    def writeback(u_ref):
        for j in range(ROWS_PT // WB):
            r0 = sid * ROWS_PT + j * WB

            @pl.when(r0 < NN)
            def _():
                pltpu.async_copy(
                    acc.at[pl.ds(r0, WB)], u_ref.at[pl.ds(r0, WB)], gsem)

        for j in range(ROWS_PT // WB):
            r0 = sid * ROWS_PT + j * WB

            @pl.when(r0 < NN)
            def _():
                pltpu.make_async_copy(
                    acc.at[pl.ds(r0, WB)], u_ref.at[pl.ds(r0, WB)], gsem).wait()

    @pl.when(cid == 0)
    def _():
        writeback(u_lo)

    @pl.when(cid == 1)
    def _():
        writeback(u_hi)


@functools.partial(
    pl.kernel,
    mesh=_MESH,
    out_type=jax.ShapeDtypeStruct((NG, D), _f32),
    scratch_types=[
        pltpu.VMEM((CH,), jnp.int32),
        pltpu.VMEM((CH, D), _f32),
    ],
    compiler_params=_SC_CP,
)
def _gath(big, cat_idx, out, idx_v, rows_v):
    cid = lax.axis_index("c")
    sid = lax.axis_index("s")
    base = (cid * NS + sid) * GPT

    @pl.loop(0, GCH)
    def _(c):
        off = base + c * CH
        pltpu.sync_copy(cat_idx.at[pl.ds(off, CH)], idx_v)
        pltpu.sync_copy(big.at[idx_v], rows_v)
        pltpu.sync_copy(rows_v, out.at[pl.ds(off, CH)])


_RB = 2000


def _s0_body(deg_ref, e_ref, tlo_ref, thi_ref, a_ref, asq_ref):
    dg = jnp.maximum(deg_ref[...], 1.0)
    asq = 1.0 / dg
    a = lax.rsqrt(dg)
    a_ref[...] = a
    asq_ref[...] = asq
    t = e_ref[...] * a
    tlo_ref[...] = t[:, :HD]
    thi_ref[...] = t[:, HD:]


def _scale_init(deg, e0):
    return pl.pallas_call(
        _s0_body,
        grid=(NN // _RB,),
        in_specs=[
            pl.BlockSpec((_RB, 1), lambda i: (i, 0)),
            pl.BlockSpec((_RB, D), lambda i: (i, 0)),
        ],
        out_specs=[
            pl.BlockSpec((_RB, HD), lambda i: (i, 0)),
            pl.BlockSpec((_RB, HD), lambda i: (i, 0)),
            pl.BlockSpec((_RB, 1), lambda i: (i, 0)),
            pl.BlockSpec((_RB, 1), lambda i: (i, 0)),
        ],
        out_shape=[
            jax.ShapeDtypeStruct((NN, HD), _f32),
            jax.ShapeDtypeStruct((NN, HD), _f32),
            jax.ShapeDtypeStruct((NN, 1), _f32),
            jax.ShapeDtypeStruct((NN, 1), _f32),
        ],
    )(deg, e0)


def _smid_body(ulo_ref, uhi_ref, asq_ref, tlo_ref, thi_ref):
    asq = asq_ref[...]
    tlo_ref[...] = ulo_ref[...] * asq
    thi_ref[...] = uhi_ref[...] * asq


def _scale_mid(u_lo, u_hi, asq):
    h = pl.BlockSpec((_RB, HD), lambda i: (i, 0))
    return pl.pallas_call(
        _smid_body,
        grid=(NN // _RB,),
        in_specs=[h, h, pl.BlockSpec((_RB, 1), lambda i: (i, 0))],
        out_specs=[h, h],
        out_shape=[
            jax.ShapeDtypeStruct((NN, HD), _f32),
            jax.ShapeDtypeStruct((NN, HD), _f32),
        ],
    )(u_lo, u_hi, asq)


def _mean_body(e_ref, l1, h1, l2, h2, l3, h3, a_ref, o_ref):
    a = a_ref[...]
    slo = l1[...] + l2[...] + l3[...]
    shi = h1[...] + h2[...] + h3[...]
    o_ref[:, :HD] = 0.25 * (e_ref[:, :HD] + a * slo)
    o_ref[:, HD:] = 0.25 * (e_ref[:, HD:] + a * shi)


def _mean(e0, us, a):
    h = pl.BlockSpec((_RB, HD), lambda i: (i, 0))
    f = pl.BlockSpec((_RB, D), lambda i: (i, 0))
    return pl.pallas_call(
        _mean_body,
        grid=(NN // _RB,),
        in_specs=[f, h, h, h, h, h, h, pl.BlockSpec((_RB, 1), lambda i: (i, 0))],
        out_specs=f,
        out_shape=jax.ShapeDtypeStruct((NN, D), _f32),
    )(e0, *us, a)


_BB = 512


def _score_body(u_ref, p_ref, n_ref, uw_ref, pw_ref, nw_ref,
                ps_ref, ns_ref, reg_ref):
    i = pl.program_id(0)
    u = u_ref[...]
    ps_ref[...] = jnp.sum(u * p_ref[...], axis=1, keepdims=True)
    ns_ref[...] = jnp.sum(u[:, None, :] * n_ref[...], axis=-1)
    part = (jnp.sum(uw_ref[...] ** 2) + jnp.sum(pw_ref[...] ** 2)
            + jnp.sum(nw_ref[...] ** 2)) * (1.0 / B)

    @pl.when(i == 0)
    def _():
        reg_ref[...] = jnp.zeros((1, 1), _f32)

    reg_ref[...] = reg_ref[...] + part


def _score(uvec, pvec, nvec, uw, pw, nw):
    v2 = pl.BlockSpec((_BB, D), lambda i: (i, 0))
    v3 = pl.BlockSpec((_BB, NEG, D), lambda i: (i, 0, 0))
    return pl.pallas_call(
        _score_body,
        grid=(B // _BB,),
        in_specs=[v2, v2, v3, v2, v2, v3],
        out_specs=[
            pl.BlockSpec((_BB, 1), lambda i: (i, 0)),
            pl.BlockSpec((_BB, NEG), lambda i: (i, 0)),
            pl.BlockSpec((1, 1), lambda i: (0, 0)),
        ],
        out_shape=[
            jax.ShapeDtypeStruct((B, 1), _f32),
            jax.ShapeDtypeStruct((B, NEG), _f32),
            jax.ShapeDtypeStruct((1, 1), _f32),
        ],
    )(uvec, pvec, nvec, uw, pw, nw)


def kernel(U_weight, I_weight, edge_vals, user, pos_item, neg_item,
           edge_row, edge_col):
    del edge_vals
    all_emb0 = jnp.concatenate([U_weight, I_weight], axis=0)

    row2 = edge_row.reshape(NW, EPT)
    col2 = edge_col.reshape(NW, EPT)
    pad_row = jnp.full((NW, PAD_EPT - EPT), SCRAP, jnp.int32)
    pad_col = jnp.zeros((NW, PAD_EPT - EPT), jnp.int32)
    rowp = jnp.concatenate([row2, pad_row], axis=1).reshape(-1)
    colp = jnp.concatenate([col2, pad_col], axis=1).reshape(-1)
    idxp = jnp.stack(
        [colp.reshape(-1, CH), rowp.reshape(-1, CH)], axis=1)

    deg = _hist(rowp)
    t_lo, t_hi, a, asq = _scale_init(deg.reshape(NN, 1), all_emb0)
    u1 = _prop(idxp, t_lo, t_hi)
    t_lo, t_hi = _scale_mid(u1[0], u1[1], asq)
    u2 = _prop(idxp, t_lo, t_hi)
    t_lo, t_hi = _scale_mid(u2[0], u2[1], asq)
    u3 = _prop(idxp, t_lo, t_hi)
    light = _mean(all_emb0, (u1[0], u1[1], u2[0], u2[1], u3[0], u3[1]), a)

    big = jnp.concatenate([light, all_emb0], axis=0)
    negf = neg_item.reshape(-1)
    cat_idx = jnp.concatenate([
        user, pos_item + USER, negf + USER,
        user + NN, pos_item + NN + USER, negf + NN + USER,
    ]).astype(jnp.int32)
    g = _gath(big, cat_idx)

    uvec = g[0:B]
    pvec = g[B:2 * B]
    nvec = g[2 * B:2 * B + B * NEG].reshape(B, NEG, D)
    o = 2 * B + B * NEG
    uw = g[o:o + B]
    pw = g[o + B:o + 2 * B]
    nw = g[o + 2 * B:o + 2 * B + B * NEG].reshape(B, NEG, D)

    ps, ns, reg = _score(uvec, pvec, nvec, uw, pw, nw)
    return ps, ns, reg[0, 0]


# speedup vs baseline: 4.9549x; 1.0182x over previous
import dataclasses
import functools

import jax
import jax.numpy as jnp
from jax import lax
from jax.experimental import pallas as pl
from jax.experimental.pallas import tpu as pltpu
from jax.experimental.pallas import tpu_sc as plsc

USER = 20000
ITEM = 30000
NN = USER + ITEM
D = 64
HD = D // 2
E = 800000
B = 4096
NEG = 16

NC, NS = 2, 16
NW = NC * NS
EPT = E // NW
CH = 128
PAD_EPT = 25728
E_PAD = NW * PAD_EPT
HCHUNK = PAD_EPT // CH
PCHUNK = 2 * HCHUNK

ACC_ROWS = 51200
ROWS_PT = ACC_ROWS // NS
SCRAP = ACC_ROWS - 1
WB = 80

NG = 2 * (B + B + B * NEG)
GPT = NG // NW
GCH = GPT // CH

_MESH = plsc.VectorSubcoreMesh(core_axis_name="c", subcore_axis_name="s")
_f32 = jnp.float32

_SC_CP = pltpu.CompilerParams()
if "needs_layout_passes" in pltpu.CompilerParams.__dataclass_fields__:
    _SC_CP = dataclasses.replace(_SC_CP, needs_layout_passes=False)
if "use_tc_tiling_on_sc" in pltpu.CompilerParams.__dataclass_fields__:
    _SC_CP = dataclasses.replace(_SC_CP, use_tc_tiling_on_sc=False)


def _zero16():
    return jnp.zeros((16,), _f32)


@functools.partial(
    pl.kernel,
    mesh=_MESH,
    out_type=jax.ShapeDtypeStruct((NN,), _f32),
    scratch_types=[
        pltpu.VMEM((CH,), jnp.int32),
        pltpu.VMEM((ACC_ROWS,), _f32),
        pltpu.VMEM((ROWS_PT,), _f32),
        pltpu.VMEM((ROWS_PT,), _f32),
        pltpu.VMEM_SHARED((NS, ACC_ROWS), _f32),
    ],
    compiler_params=_SC_CP,
)
def _hist(rowp, deg_out, row_v, hist_v, part_v, sum_v, stage):
    cid = lax.axis_index("c")
    sid = lax.axis_index("s")
    ebase = (cid * NS + sid) * PAD_EPT

    @pl.loop(0, ACC_ROWS // 16)
    def _(i):
        hist_v[pl.ds(i * 16, 16)] = _zero16()

    @pl.loop(0, HCHUNK)
    def _(c):
        pltpu.sync_copy(rowp.at[pl.ds(ebase + c * CH, CH)], row_v)
        for j in range(CH // 16):
            idx16 = row_v[pl.ds(j * 16, 16)]
            plsc.addupdate_scatter(hist_v, [idx16], jnp.ones((16,), _f32))

    pltpu.sync_copy(hist_v, stage.at[sid])
    plsc.subcore_barrier()

    @pl.loop(0, ROWS_PT // 16)
    def _(i):
        sum_v[pl.ds(i * 16, 16)] = _zero16()

    @pl.loop(0, NS)
    def _(p):
        pltpu.sync_copy(stage.at[p, pl.ds(sid * ROWS_PT, ROWS_PT)], part_v)

        @pl.loop(0, ROWS_PT // 16)
        def _(i):
            sl = pl.ds(i * 16, 16)
            sum_v[sl] = sum_v[sl] + part_v[sl]

    lo = cid * USER
    hi = jnp.where(cid == 0, USER, NN)

    @pl.loop(0, ROWS_PT // WB)
    def _(j):
        r0 = sid * ROWS_PT + j * WB

        @pl.when(jnp.logical_and(r0 >= lo, r0 < hi))
        def _():
            pltpu.sync_copy(sum_v.at[pl.ds(j * WB, WB)],
                            deg_out.at[pl.ds(r0, WB)])


NB = 3
NGRP = PCHUNK // NB


@functools.partial(
    pl.kernel,
    mesh=_MESH,
    out_type=jax.ShapeDtypeStruct((2 * NN, HD), _f32),
    scratch_types=[
        pltpu.VMEM((2, NB, 2, CH), jnp.int32),
        pltpu.VMEM((2, NB, CH, HD), _f32),
        pltpu.VMEM_SHARED((ACC_ROWS, HD), _f32),
        pltpu.SemaphoreType.DMA,
        pltpu.SemaphoreType.DMA,
        pltpu.SemaphoreType.DMA,
    ],
    compiler_params=_SC_CP,
)
def _prop(idxp, t_st, zeros, u_st, ibuf, rbuf, acc, isem, gsem, ssem):
    cid = lax.axis_index("c")
    sid = lax.axis_index("s")
    cbase = (cid * NW // 2 + sid) * PCHUNK

    @pl.loop(0, ROWS_PT // CH)
    def _(j):
        pltpu.async_copy(zeros, acc.at[pl.ds(sid * ROWS_PT + j * CH, CH)],
                         ssem)

    @pl.loop(0, ROWS_PT // CH)
    def _(j):
        pltpu.make_async_copy(
            zeros, acc.at[pl.ds(sid * ROWS_PT + j * CH, CH)], ssem).wait()

    plsc.subcore_barrier()

    for b in range(NB):
        pltpu.async_copy(idxp.at[cbase + b], ibuf.at[0, b], isem)

    @pl.loop(0, NGRP + 1)
    def _(g):
        p = lax.rem(g, 2)
        q = lax.rem(g + 1, 2)

        @pl.when(g < NGRP)
        def _():
            for b in range(NB):
                pltpu.make_async_copy(
                    idxp.at[cbase], ibuf.at[p, b], isem).wait()
            for b in range(NB):
                pltpu.async_copy(
                    t_st.at[ibuf.at[p, b, 0]], rbuf.at[p, b], gsem)

        @pl.when(g > 0)
        def _():
            for b in range(NB):
                pltpu.make_async_copy(
                    t_st.at[pl.ds(0, CH)], rbuf.at[0, 0], ssem).wait()

        @pl.when(g + 1 < NGRP)
        def _():
            c0 = cbase + (g + 1) * NB
            for b in range(NB):
                pltpu.async_copy(idxp.at[c0 + b], ibuf.at[q, b], isem)

        @pl.when(g < NGRP)
        def _():
            for b in range(NB):
                pltpu.make_async_copy(
                    t_st.at[pl.ds(0, CH)], rbuf.at[0, 0], gsem).wait()
            for b in range(NB):
                pltpu.async_copy(rbuf.at[p, b], acc.at[ibuf.at[p, b, 1]],
                                 ssem, add=True)

    plsc.subcore_barrier()

    ubase = cid * NN

    @pl.loop(0, ROWS_PT // WB)
    def _(j):
        r0 = sid * ROWS_PT + j * WB

        @pl.when(r0 < NN)
        def _():
            pltpu.async_copy(
                acc.at[pl.ds(r0, WB)], u_st.at[pl.ds(ubase + r0, WB)], gsem)

    @pl.loop(0, ROWS_PT // WB)
    def _(j):
        r0 = sid * ROWS_PT + j * WB

        @pl.when(r0 < NN)
        def _():
            pltpu.make_async_copy(
                acc.at[pl.ds(r0, WB)], u_st.at[pl.ds(ubase + r0, WB)],
                gsem).wait()


@functools.partial(
    pl.kernel,
    mesh=_MESH,
    out_type=jax.ShapeDtypeStruct((NG, D), _f32),
    scratch_types=[
        pltpu.VMEM((CH,), jnp.int32),
        pltpu.VMEM((CH, D), _f32),
    ],
    compiler_params=_SC_CP,
)
def _gath(big, cat_idx, out, idx_v, rows_v):
    cid = lax.axis_index("c")
    sid = lax.axis_index("s")
    base = (cid * NS + sid) * GPT

    @pl.loop(0, GCH)
    def _(c):
        off = base + c * CH
        pltpu.sync_copy(cat_idx.at[pl.ds(off, CH)], idx_v)
        pltpu.sync_copy(big.at[idx_v], rows_v)
        pltpu.sync_copy(rows_v, out.at[pl.ds(off, CH)])


_RB = 2000


_NRB = NN // _RB

_ST3 = pl.BlockSpec((2, _RB, HD), lambda i: (0, i, 0))
_FULL = pl.BlockSpec((_RB, D), lambda i: (i, 0))
_COL1 = pl.BlockSpec((_RB, 1), lambda i: (i, 0))


def _s0_body(deg_ref, e_ref, t_ref, a_ref, asq_ref):
    dg = jnp.maximum(deg_ref[...], 1.0)
    asq = 1.0 / dg
    a = lax.rsqrt(dg)
    a_ref[...] = a
    asq_ref[...] = asq
    e = e_ref[...]
    t_ref[0] = e[:, :HD] * a
    t_ref[1] = e[:, HD:] * a


def _scale_init(deg, e0):
    return pl.pallas_call(
        _s0_body,
        grid=(_NRB,),
        in_specs=[_COL1, _FULL],
        out_specs=[_ST3, _COL1, _COL1],
        out_shape=[
            jax.ShapeDtypeStruct((2, NN, HD), _f32),
            jax.ShapeDtypeStruct((NN, 1), _f32),
            jax.ShapeDtypeStruct((NN, 1), _f32),
        ],
    )(deg, e0)


def _smid_body(u_ref, asq_ref, t_ref):
    t_ref[...] = u_ref[...] * asq_ref[...]


def _scale_mid(u_st, asq):
    return pl.pallas_call(
        _smid_body,
        grid=(_NRB,),
        in_specs=[_ST3, _COL1],
        out_specs=_ST3,
        out_shape=jax.ShapeDtypeStruct((2, NN, HD), _f32),
    )(u_st, asq)


def _mean_body(e_ref, u1_ref, u2_ref, u3_ref, a_ref, o_ref):
    s = u1_ref[...] + u2_ref[...] + u3_ref[...]
    a = a_ref[...]
    e = e_ref[...]
    o_ref[:, :HD] = 0.25 * (e[:, :HD] + a * s[0])
    o_ref[:, HD:] = 0.25 * (e[:, HD:] + a * s[1])


def _mean(e0, u1, u2, u3, a):
    return pl.pallas_call(
        _mean_body,
        grid=(_NRB,),
        in_specs=[_FULL, _ST3, _ST3, _ST3, _COL1],
        out_specs=_FULL,
        out_shape=jax.ShapeDtypeStruct((NN, D), _f32),
    )(e0, u1, u2, u3, a)


_BB = 512


def _score_body(u_ref, p_ref, n_ref, uw_ref, pw_ref, nw_ref,
                ps_ref, ns_ref, reg_ref):
    i = pl.program_id(0)
    u = u_ref[...]
    ps_ref[...] = jnp.sum(u * p_ref[...], axis=1, keepdims=True)
    ns_ref[...] = jnp.sum(u[:, None, :] * n_ref[...], axis=-1)
    part = (jnp.sum(uw_ref[...] ** 2) + jnp.sum(pw_ref[...] ** 2)
            + jnp.sum(nw_ref[...] ** 2)) * (1.0 / B)

    @pl.when(i == 0)
    def _():
        reg_ref[...] = jnp.zeros((1, 1), _f32)

    reg_ref[...] = reg_ref[...] + part


def _score(uvec, pvec, nvec, uw, pw, nw):
    v2 = pl.BlockSpec((_BB, D), lambda i: (i, 0))
    v3 = pl.BlockSpec((_BB, NEG, D), lambda i: (i, 0, 0))
    return pl.pallas_call(
        _score_body,
        grid=(B // _BB,),
        in_specs=[v2, v2, v3, v2, v2, v3],
        out_specs=[
            pl.BlockSpec((_BB, 1), lambda i: (i, 0)),
            pl.BlockSpec((_BB, NEG), lambda i: (i, 0)),
            pl.BlockSpec((1, 1), lambda i: (0, 0)),
        ],
        out_shape=[
            jax.ShapeDtypeStruct((B, 1), _f32),
            jax.ShapeDtypeStruct((B, NEG), _f32),
            jax.ShapeDtypeStruct((1, 1), _f32),
        ],
    )(uvec, pvec, nvec, uw, pw, nw)


def kernel(U_weight, I_weight, edge_vals, user, pos_item, neg_item,
           edge_row, edge_col):
    del edge_vals
    all_emb0 = jnp.concatenate([U_weight, I_weight], axis=0)

    row2 = edge_row.reshape(NW, EPT)
    col2 = edge_col.reshape(NW, EPT)
    pad_row = jnp.full((NW, PAD_EPT - EPT), SCRAP, jnp.int32)
    pad_col = jnp.zeros((NW, PAD_EPT - EPT), jnp.int32)
    rowp = jnp.concatenate([row2, pad_row], axis=1).reshape(-1)
    colp = jnp.concatenate([col2, pad_col], axis=1).reshape(-1)
    colch = colp.reshape(-1, CH)
    rowch = rowp.reshape(-1, CH)
    idxp = jnp.concatenate([
        jnp.stack([colch, rowch], axis=1),
        jnp.stack([colch + NN, rowch], axis=1),
    ], axis=0)

    deg = _hist(rowp)
    t_st, a, asq = _scale_init(deg.reshape(NN, 1), all_emb0)
    zblk = jnp.zeros((CH, HD), _f32)
    u1 = _prop(idxp, t_st.reshape(2 * NN, HD), zblk)
    t_st = _scale_mid(u1.reshape(2, NN, HD), asq)
    u2 = _prop(idxp, t_st.reshape(2 * NN, HD), zblk)
    t_st = _scale_mid(u2.reshape(2, NN, HD), asq)
    u3 = _prop(idxp, t_st.reshape(2 * NN, HD), zblk)
    light = _mean(all_emb0, u1.reshape(2, NN, HD), u2.reshape(2, NN, HD),
                  u3.reshape(2, NN, HD), a)

    big = jnp.concatenate([light, all_emb0], axis=0)
    negf = neg_item.reshape(-1)
    cat_idx = jnp.concatenate([
        user, pos_item + USER, negf + USER,
        user + NN, pos_item + NN + USER, negf + NN + USER,
    ]).astype(jnp.int32)
    g = _gath(big, cat_idx)

    uvec = g[0:B]
    pvec = g[B:2 * B]
    nvec = g[2 * B:2 * B + B * NEG].reshape(B, NEG, D)
    o = 2 * B + B * NEG
    uw = g[o:o + B]
    pw = g[o + B:o + 2 * B]
    nw = g[o + 2 * B:o + 2 * B + B * NEG].reshape(B, NEG, D)

    ps, ns, reg = _score(uvec, pvec, nvec, uw, pw, nw)
    return ps, ns, reg[0, 0]


# speedup vs baseline: 4.9944x; 1.0080x over previous
import dataclasses
import functools

import jax
import jax.numpy as jnp
from jax import lax
from jax.experimental import pallas as pl
from jax.experimental.pallas import tpu as pltpu
from jax.experimental.pallas import tpu_sc as plsc

USER = 20000
ITEM = 30000
NN = USER + ITEM
D = 64
HD = D // 2
E = 800000
B = 4096
NEG = 16

NC, NS = 2, 16
NW = NC * NS
EPT = E // NW
CH = 128
PAD_EPT = 25728
E_PAD = NW * PAD_EPT
HCHUNK = PAD_EPT // CH
PCHUNK = 2 * HCHUNK

ACC_ROWS = 51200
ROWS_PT = ACC_ROWS // NS
SCRAP = ACC_ROWS - 1
WB = 80

NG = 2 * (B + B + B * NEG)
GPT = NG // NW
GCH = GPT // CH

_MESH = plsc.VectorSubcoreMesh(core_axis_name="c", subcore_axis_name="s")
_f32 = jnp.float32

_SC_CP = pltpu.CompilerParams()
if "needs_layout_passes" in pltpu.CompilerParams.__dataclass_fields__:
    _SC_CP = dataclasses.replace(_SC_CP, needs_layout_passes=False)
if "use_tc_tiling_on_sc" in pltpu.CompilerParams.__dataclass_fields__:
    _SC_CP = dataclasses.replace(_SC_CP, use_tc_tiling_on_sc=False)


def _zero16():
    return jnp.zeros((16,), _f32)


@functools.partial(
    pl.kernel,
    mesh=_MESH,
    out_type=jax.ShapeDtypeStruct((NN,), _f32),
    scratch_types=[
        pltpu.VMEM((CH,), jnp.int32),
        pltpu.VMEM((ACC_ROWS,), _f32),
        pltpu.VMEM((ROWS_PT,), _f32),
        pltpu.VMEM((ROWS_PT,), _f32),
        pltpu.VMEM_SHARED((NS, ACC_ROWS), _f32),
    ],
    compiler_params=_SC_CP,
)
def _hist(rowp, deg_out, row_v, hist_v, part_v, sum_v, stage):
    cid = lax.axis_index("c")
    sid = lax.axis_index("s")
    ebase = (cid * NS + sid) * PAD_EPT

    @pl.loop(0, ACC_ROWS // 16)
    def _(i):
        hist_v[pl.ds(i * 16, 16)] = _zero16()

    @pl.loop(0, HCHUNK)
    def _(c):
        pltpu.sync_copy(rowp.at[pl.ds(ebase + c * CH, CH)], row_v)
        for j in range(CH // 16):
            idx16 = row_v[pl.ds(j * 16, 16)]
            plsc.addupdate_scatter(hist_v, [idx16], jnp.ones((16,), _f32))

    pltpu.sync_copy(hist_v, stage.at[sid])
    plsc.subcore_barrier()

    @pl.loop(0, ROWS_PT // 16)
    def _(i):
        sum_v[pl.ds(i * 16, 16)] = _zero16()

    @pl.loop(0, NS)
    def _(p):
        pltpu.sync_copy(stage.at[p, pl.ds(sid * ROWS_PT, ROWS_PT)], part_v)

        @pl.loop(0, ROWS_PT // 16)
        def _(i):
            sl = pl.ds(i * 16, 16)
            sum_v[sl] = sum_v[sl] + part_v[sl]

    lo = cid * USER
    hi = jnp.where(cid == 0, USER, NN)

    @pl.loop(0, ROWS_PT // WB)
    def _(j):
        r0 = sid * ROWS_PT + j * WB

        @pl.when(jnp.logical_and(r0 >= lo, r0 < hi))
        def _():
            pltpu.sync_copy(sum_v.at[pl.ds(j * WB, WB)],
                            deg_out.at[pl.ds(r0, WB)])


NB = 3
NGRP = PCHUNK // NB
SP = ACC_ROWS


@functools.partial(
    pl.kernel,
    mesh=_MESH,
    out_type=[
        jax.ShapeDtypeStruct((2 * SP, HD), _f32),
        jax.ShapeDtypeStruct((2 * SP, HD), _f32),
        jax.ShapeDtypeStruct((2 * SP, HD), _f32),
    ],
    scratch_types=[
        pltpu.VMEM((2, NB, 2, CH), jnp.int32),
        pltpu.VMEM((2, NB, CH, HD), _f32),
        pltpu.VMEM_SHARED((ACC_ROWS, HD), _f32),
        pltpu.SemaphoreType.DMA,
        pltpu.SemaphoreType.DMA,
        pltpu.SemaphoreType.DMA,
    ],
    compiler_params=_SC_CP,
)
def _prop3(idxp, t0, asqx, zeros, t1, t2, t3, ibuf, rbuf, acc,
           isem, gsem, ssem):
    cid = lax.axis_index("c")
    sid = lax.axis_index("s")
    cbase = (cid * NW // 2 + sid) * PCHUNK
    ubase = cid * SP

    def layer(t_src, t_dst):
        @pl.loop(0, ROWS_PT // CH)
        def _(j):
            pltpu.async_copy(zeros, acc.at[pl.ds(sid * ROWS_PT + j * CH, CH)],
                             ssem)

        @pl.loop(0, ROWS_PT // CH)
        def _(j):
            pltpu.make_async_copy(
                zeros, acc.at[pl.ds(sid * ROWS_PT + j * CH, CH)], ssem).wait()

        plsc.subcore_barrier()

        for b in range(NB):
            pltpu.async_copy(idxp.at[cbase + b], ibuf.at[0, b], isem)

        @pl.loop(0, NGRP + 1)
        def _(g):
            p = lax.rem(g, 2)
            q = lax.rem(g + 1, 2)

            @pl.when(g < NGRP)
            def _():
                for b in range(NB):
                    pltpu.make_async_copy(
                        idxp.at[cbase], ibuf.at[p, b], isem).wait()
                for b in range(NB):
                    pltpu.async_copy(
                        t_src.at[ibuf.at[p, b, 0]], rbuf.at[p, b], gsem)

            @pl.when(g > 0)
            def _():
                for b in range(NB):
                    pltpu.make_async_copy(
                        t_src.at[pl.ds(0, CH)], rbuf.at[0, 0], ssem).wait()

            @pl.when(g + 1 < NGRP)
            def _():
                c0 = cbase + (g + 1) * NB
                for b in range(NB):
                    pltpu.async_copy(idxp.at[c0 + b], ibuf.at[q, b], isem)

            @pl.when(g < NGRP)
            def _():
                for b in range(NB):
                    pltpu.make_async_copy(
                        t_src.at[pl.ds(0, CH)], rbuf.at[0, 0], gsem).wait()
                for b in range(NB):
                    pltpu.async_copy(rbuf.at[p, b], acc.at[ibuf.at[p, b, 1]],
                                     ssem, add=True)

        plsc.subcore_barrier()

        @pl.loop(0, ROWS_PT // CH)
        def _(j):
            r0 = sid * ROWS_PT + j * CH
            pltpu.sync_copy(acc.at[pl.ds(r0, CH)], rbuf.at[0, 0])
            pltpu.sync_copy(asqx.at[pl.ds(r0, CH)], rbuf.at[0, 1])

            @pl.loop(0, CH)
            def _(r):
                for k in range(HD // 16):
                    sl = pl.ds(k * 16, 16)
                    rbuf[0, 0, r, sl] = rbuf[0, 0, r, sl] * rbuf[0, 1, r, sl]

            pltpu.sync_copy(rbuf.at[0, 0], t_dst.at[pl.ds(ubase + r0, CH)])

        plsc.subcore_barrier()

    layer(t0, t1)
    layer(t1, t2)
    layer(t2, t3)


@functools.partial(
    pl.kernel,
    mesh=_MESH,
    out_type=jax.ShapeDtypeStruct((NG, D), _f32),
    scratch_types=[
        pltpu.VMEM((CH,), jnp.int32),
        pltpu.VMEM((CH, D), _f32),
    ],
    compiler_params=_SC_CP,
)
def _gath(big, cat_idx, out, idx_v, rows_v):
    cid = lax.axis_index("c")
    sid = lax.axis_index("s")
    base = (cid * NS + sid) * GPT

    @pl.loop(0, GCH)
    def _(c):
        off = base + c * CH
        pltpu.sync_copy(cat_idx.at[pl.ds(off, CH)], idx_v)
        pltpu.sync_copy(big.at[idx_v], rows_v)
        pltpu.sync_copy(rows_v, out.at[pl.ds(off, CH)])


_RB = 2000


_NRB = NN // _RB

_ST3 = pl.BlockSpec((2, _RB, HD), lambda i: (0, i, 0))
_SPH = pl.BlockSpec((_RB, HD), lambda i: (i, 0))
_FULL = pl.BlockSpec((_RB, D), lambda i: (i, 0))
_COL1 = pl.BlockSpec((_RB, 1), lambda i: (i, 0))


def _s0_body(deg_ref, e_ref, t_ref, asqx_ref, sd_ref):
    dg = jnp.maximum(deg_ref[...], 1.0)
    a = lax.rsqrt(dg)
    sd_ref[...] = jnp.sqrt(dg)
    asqx_ref[...] = jnp.broadcast_to(1.0 / dg, (_RB, HD))
    e = e_ref[...]
    t_ref[0] = e[:, :HD] * a
    t_ref[1] = e[:, HD:] * a


def _scale_init(deg, e0):
    return pl.pallas_call(
        _s0_body,
        grid=(_NRB,),
        in_specs=[_COL1, _FULL],
        out_specs=[_ST3, _SPH, _COL1],
        out_shape=[
            jax.ShapeDtypeStruct((2, SP, HD), _f32),
            jax.ShapeDtypeStruct((SP, HD), _f32),
            jax.ShapeDtypeStruct((NN, 1), _f32),
        ],
    )(deg, e0)


def _mean_body(e_ref, t1_ref, t2_ref, t3_ref, sd_ref, o_ref):
    s = t1_ref[...] + t2_ref[...] + t3_ref[...]
    sd = sd_ref[...]
    e = e_ref[...]
    o_ref[:, :HD] = 0.25 * (e[:, :HD] + sd * s[0])
    o_ref[:, HD:] = 0.25 * (e[:, HD:] + sd * s[1])


def _mean(e0, t1, t2, t3, sd):
    return pl.pallas_call(
        _mean_body,
        grid=(_NRB,),
        in_specs=[_FULL, _ST3, _ST3, _ST3, _COL1],
        out_specs=_FULL,
        out_shape=jax.ShapeDtypeStruct((NN, D), _f32),
    )(e0, t1, t2, t3, sd)


_BB = 512


def _score_body(u_ref, p_ref, n_ref, uw_ref, pw_ref, nw_ref,
                ps_ref, ns_ref, reg_ref):
    i = pl.program_id(0)
    u = u_ref[...]
    ps_ref[...] = jnp.sum(u * p_ref[...], axis=1, keepdims=True)
    ns_ref[...] = jnp.sum(u[:, None, :] * n_ref[...], axis=-1)
    part = (jnp.sum(uw_ref[...] ** 2) + jnp.sum(pw_ref[...] ** 2)
            + jnp.sum(nw_ref[...] ** 2)) * (1.0 / B)

    @pl.when(i == 0)
    def _():
        reg_ref[...] = jnp.zeros((1, 1), _f32)

    reg_ref[...] = reg_ref[...] + part


def _score(uvec, pvec, nvec, uw, pw, nw):
    v2 = pl.BlockSpec((_BB, D), lambda i: (i, 0))
    v3 = pl.BlockSpec((_BB, NEG, D), lambda i: (i, 0, 0))
    return pl.pallas_call(
        _score_body,
        grid=(B // _BB,),
        in_specs=[v2, v2, v3, v2, v2, v3],
        out_specs=[
            pl.BlockSpec((_BB, 1), lambda i: (i, 0)),
            pl.BlockSpec((_BB, NEG), lambda i: (i, 0)),
            pl.BlockSpec((1, 1), lambda i: (0, 0)),
        ],
        out_shape=[
            jax.ShapeDtypeStruct((B, 1), _f32),
            jax.ShapeDtypeStruct((B, NEG), _f32),
            jax.ShapeDtypeStruct((1, 1), _f32),
        ],
    )(uvec, pvec, nvec, uw, pw, nw)


def kernel(U_weight, I_weight, edge_vals, user, pos_item, neg_item,
           edge_row, edge_col):
    del edge_vals
    all_emb0 = jnp.concatenate([U_weight, I_weight], axis=0)

    row2 = edge_row.reshape(NW, EPT)
    col2 = edge_col.reshape(NW, EPT)
    pad_row = jnp.full((NW, PAD_EPT - EPT), SCRAP, jnp.int32)
    pad_col = jnp.zeros((NW, PAD_EPT - EPT), jnp.int32)
    rowp = jnp.concatenate([row2, pad_row], axis=1).reshape(-1)
    colp = jnp.concatenate([col2, pad_col], axis=1).reshape(-1)
    colch = colp.reshape(-1, CH)
    rowch = rowp.reshape(-1, CH)
    idxp = jnp.concatenate([
        jnp.stack([colch, rowch], axis=1),
        jnp.stack([colch + SP, rowch], axis=1),
    ], axis=0)

    deg = _hist(rowp)
    t0, asqx, sd = _scale_init(deg.reshape(NN, 1), all_emb0)
    zblk = jnp.zeros((CH, HD), _f32)
    t1, t2, t3 = _prop3(idxp, t0.reshape(2 * SP, HD), asqx, zblk)
    light = _mean(all_emb0, t1.reshape(2, SP, HD), t2.reshape(2, SP, HD),
                  t3.reshape(2, SP, HD), sd)

    big = jnp.concatenate([light, all_emb0], axis=0)
    negf = neg_item.reshape(-1)
    cat_idx = jnp.concatenate([
        user, pos_item + USER, negf + USER,
        user + NN, pos_item + NN + USER, negf + NN + USER,
    ]).astype(jnp.int32)
    g = _gath(big, cat_idx)

    uvec = g[0:B]
    pvec = g[B:2 * B]
    nvec = g[2 * B:2 * B + B * NEG].reshape(B, NEG, D)
    o = 2 * B + B * NEG
    uw = g[o:o + B]
    pw = g[o + B:o + 2 * B]
    nw = g[o + 2 * B:o + 2 * B + B * NEG].reshape(B, NEG, D)

    ps, ns, reg = _score(uvec, pvec, nvec, uw, pw, nw)
    return ps, ns, reg[0, 0]
